# baseline (speedup 1.0000x reference)
import jax
import jax.numpy as jnp
from jax import lax
from jax.experimental import pallas as pl
from jax.experimental.pallas import tpu as pltpu
from jax.experimental.pallas import tpu_sc as plsc

_N = 10000
_D = 128
_E = 320000
_CHUNK = 80
_NCHUNK = _E // _CHUNK
_NBUF = 4
_NSUB = 16
_KTILE = _NCHUNK // _NSUB
_ROWS_PER_TILE = 624
_TAIL_BASE = _NSUB * _ROWS_PER_TILE
_TAIL = _N - _TAIL_BASE
_COPIES = [(i * 80, 80) for i in range(7)] + [(560, 64)]


def _sc_body(x_hbm, src0, dst0, src1, dst1, h0_hbm, h1_hbm,
             idx_src, idx_dst, rows, acc, lsrc_sem, ldst_sem, g_sem, s_sem):
    c = lax.axis_index("c")
    s = lax.axis_index("s")

    def xfer(src_ref, dst_ref, base):
        chunks = list(_COPIES)
        loads = [pltpu.make_async_copy(src_ref.at[pl.ds(base + off, size)],
                                       rows[j % _NBUF].at[pl.ds(0, size)],
                                       g_sem[j % _NBUF])
                 for j, (off, size) in enumerate(chunks)]
        for j in range(_NBUF):
            loads[j].start()
        for j, (off, size) in enumerate(chunks):
            loads[j].wait()
            pltpu.sync_copy(rows[j % _NBUF].at[pl.ds(0, size)],
                            dst_ref.at[pl.ds(base + off, size)])
            if j + _NBUF < len(chunks):
                loads[j + _NBUF].start()

        @pl.when(s == _NSUB - 1)
        def _():
            pltpu.sync_copy(src_ref.at[pl.ds(_TAIL_BASE, _TAIL)],
                            rows[0].at[pl.ds(0, _TAIL)])
            pltpu.sync_copy(rows[0].at[pl.ds(0, _TAIL)],
                            dst_ref.at[pl.ds(_TAIL_BASE, _TAIL)])

    def run(src1d, dst1d, out_hbm):
        base = s * _ROWS_PER_TILE
        xfer(x_hbm, acc, base)
        plsc.subcore_barrier()

        def eoff(k):
            return (k * _NSUB + s) * _CHUNK

        def lsrc(k, b):
            return pltpu.make_async_copy(
                src1d.at[pl.ds(eoff(k), _CHUNK)], idx_src[b], lsrc_sem[b])

        def ldst(k, b):
            return pltpu.make_async_copy(
                dst1d.at[pl.ds(eoff(k), _CHUNK)], idx_dst[b], ldst_sem[b])

        def gath(b):
            return pltpu.make_async_copy(x_hbm.at[idx_src[b]], rows[b],
                                         g_sem[b])

        def scat(b):
            return pltpu.async_copy(rows[b], acc.at[idx_dst[b]], s_sem[b],
                                    add=True)

        def scat_wait(b):
            pltpu.make_async_copy(rows[b], acc.at[idx_dst[b]],
                                  s_sem[b]).wait()

        for b in range(_NBUF):
            lsrc(b, b).start()
        for b in range(2):
            ldst(b, b).start()
        for b in range(2):
            lsrc(b, b).wait()
            gath(b).start()

        def body(i, carry):
            for b in range(_NBUF):
                k = _NBUF * i + b
                b2 = (b + 2) % _NBUF

                @pl.when(k < _KTILE)
                def _():
                    gath(b).wait()

                    @pl.when(k < _KTILE - _NBUF)
                    def _():
                        lsrc(k + _NBUF, b).start()

                    ldst(k, b).wait()
                    scat(b)

                @pl.when(k >= 2)
                def _():
                    scat_wait(b2)

                @pl.when(k < _KTILE - 2)
                def _():
                    ldst(k + 2, b2).start()
                    lsrc(k + 2, b2).wait()
                    gath(b2).start()

            return carry

        niter = (_KTILE + 2 + _NBUF - 1) // _NBUF
        lax.fori_loop(0, niter, body, 0)
        plsc.subcore_barrier()

        xfer(acc, out_hbm, base)

    @pl.when(c == 0)
    def _():
        run(src0, dst0, h0_hbm)

    @pl.when(c == 1)
    def _():
        run(src1, dst1, h1_hbm)


@jax.jit
def _sc_segment(x, src0, dst0, src1, dst1):
    mesh = plsc.VectorSubcoreMesh(core_axis_name="c", subcore_axis_name="s")
    f = pl.kernel(
        _sc_body,
        out_type=(
            jax.ShapeDtypeStruct((_N, _D), jnp.float32),
            jax.ShapeDtypeStruct((_N, _D), jnp.float32),
        ),
        mesh=mesh,
        scratch_types=[
            [pltpu.VMEM((_CHUNK,), jnp.int32) for _ in range(_NBUF)],
            [pltpu.VMEM((_CHUNK,), jnp.int32) for _ in range(_NBUF)],
            [pltpu.VMEM((_CHUNK, _D), jnp.float32) for _ in range(_NBUF)],
            pltpu.VMEM_SHARED((_N, _D), jnp.float32),
            [pltpu.SemaphoreType.DMA for _ in range(_NBUF)],
            [pltpu.SemaphoreType.DMA for _ in range(_NBUF)],
            [pltpu.SemaphoreType.DMA for _ in range(_NBUF)],
            [pltpu.SemaphoreType.DMA for _ in range(_NBUF)],
        ],
    )
    return f(x, src0, dst0, src1, dst1)


def _mlp_body(h0, h1, W10, b10, g0, be0, W11, b11, g1, be1, W2cat, b2sum,
              out):
    def mlp1(h_ref, W1, b1, g, be):
        t = jnp.dot(h_ref[...], W1[...], preferred_element_type=jnp.float32)
        t = t + b1[...]
        m = jnp.mean(t, axis=0, keepdims=True)
        ct = t - m
        v = jnp.mean(ct * ct, axis=0, keepdims=True)
        n = ct * lax.rsqrt(v + 1e-5) * g[...] + be[...]
        return jnp.maximum(n, 0.0)

    r = jnp.concatenate((mlp1(h0, W10, b10, g0, be0),
                         mlp1(h1, W11, b11, g1, be1)), axis=1)
    out[...] = (jnp.dot(r, W2cat[...], preferred_element_type=jnp.float32)
                + b2sum[...])


@jax.jit
def _mlp(h0, h1, *params):
    return pl.pallas_call(
        _mlp_body,
        out_shape=jax.ShapeDtypeStruct((_N, _D), jnp.float32),
    )(h0, h1, *params)


def kernel(x, edge_index_e0, edge_index_e1,
           W1_e0, b1_e0, gamma_e0, beta_e0, W2_e0, b2_e0,
           W1_e1, b1_e1, gamma_e1, beta_e1, W2_e1, b2_e1):
    h0, h1 = _sc_segment(x, edge_index_e0[0], edge_index_e0[1],
                         edge_index_e1[0], edge_index_e1[1])

    p2 = lambda a: a.reshape(1, _D)
    W2cat = jnp.concatenate((W2_e0, W2_e1), axis=0)
    b2sum = p2(b2_e0 + b2_e1)
    return _mlp(h0, h1,
                W1_e0, p2(b1_e0), p2(gamma_e0), p2(beta_e0),
                W1_e1, p2(b1_e1), p2(gamma_e1), p2(beta_e1),
                W2cat, b2sum)


# speedup vs baseline: 1.1796x; 1.1796x over previous
import jax
import jax.numpy as jnp
from jax import lax
from jax.experimental import pallas as pl
from jax.experimental.pallas import tpu as pltpu
from jax.experimental.pallas import tpu_sc as plsc

_N = 10000
_D = 128
_E = 320000
_CHUNK = 128
_NCHUNK = _E // _CHUNK
_NBUF = 3
_NSUB = 16
_ROWS_PER_TILE = 624
_TAIL_BASE = _NSUB * _ROWS_PER_TILE
_TAIL = _N - _TAIL_BASE
_COPIES = [(0, 128), (128, 128), (256, 128), (384, 128), (512, 112)]


def _sc_body(x_hbm, src0, dst0, src1, dst1, h0_hbm, h1_hbm,
             idx_src, idx_dst, rows, acc, lsrc_sem, ldst_sem, g_sem):
    c = lax.axis_index("c")
    s = lax.axis_index("s")

    def xfer(src_ref, dst_ref, base):
        pltpu.sync_copy(src_ref.at[pl.ds(base, _ROWS_PER_TILE)],
                        dst_ref.at[pl.ds(base, _ROWS_PER_TILE)])

        @pl.when(s == _NSUB - 1)
        def _():
            pltpu.sync_copy(src_ref.at[pl.ds(_TAIL_BASE, _TAIL)],
                            dst_ref.at[pl.ds(_TAIL_BASE, _TAIL)])

    def run(src1d, dst1d, out_hbm):
        base = s * _ROWS_PER_TILE
        xfer(x_hbm, acc, base)
        plsc.subcore_barrier()

        def pred(k):
            return k * _NSUB + s < _NCHUNK

        def eoff(k):
            return (k * _NSUB + s) * _CHUNK

        def lsrc(k, b):
            return pltpu.make_async_copy(
                src1d.at[pl.ds(eoff(k), _CHUNK)], idx_src[b], lsrc_sem[b])

        def ldst(k, b):
            return pltpu.make_async_copy(
                dst1d.at[pl.ds(eoff(k), _CHUNK)], idx_dst[b], ldst_sem[b])

        def gath(b):
            return pltpu.make_async_copy(x_hbm.at[idx_src[b]], rows[b],
                                         g_sem[b])

        for b in range(_NBUF):
            @pl.when(pred(b))
            def _():
                lsrc(b, b).start()
                ldst(b, b).start()
        for b in range(_NBUF):
            @pl.when(pred(b))
            def _():
                lsrc(b, b).wait()
                gath(b).start()

        def body(i, carry):
            for b in range(_NBUF):
                k = _NBUF * i + b

                @pl.when(pred(k))
                def _():
                    gath(b).wait()

                    @pl.when(pred(k + _NBUF))
                    def _():
                        lsrc(k + _NBUF, b).start()

                    ldst(k, b).wait()
                    pltpu.sync_copy(rows[b], acc.at[idx_dst[b]], add=True)

                    @pl.when(pred(k + _NBUF))
                    def _():
                        ldst(k + _NBUF, b).start()
                        lsrc(k + _NBUF, b).wait()
                        gath(b).start()

            return carry

        niter = (_NCHUNK // _NSUB + 1 + _NBUF) // _NBUF
        lax.fori_loop(0, niter, body, 0)
        plsc.subcore_barrier()

        xfer(acc, out_hbm, base)

    @pl.when(c == 0)
    def _():
        run(src0, dst0, h0_hbm)

    @pl.when(c == 1)
    def _():
        run(src1, dst1, h1_hbm)


@jax.jit
def _sc_segment(x, src0, dst0, src1, dst1):
    mesh = plsc.VectorSubcoreMesh(core_axis_name="c", subcore_axis_name="s")
    f = pl.kernel(
        _sc_body,
        out_type=(
            jax.ShapeDtypeStruct((_N, _D), jnp.float32),
            jax.ShapeDtypeStruct((_N, _D), jnp.float32),
        ),
        mesh=mesh,
        scratch_types=[
            [pltpu.VMEM((_CHUNK,), jnp.int32) for _ in range(_NBUF)],
            [pltpu.VMEM((_CHUNK,), jnp.int32) for _ in range(_NBUF)],
            [pltpu.VMEM((_CHUNK, _D), jnp.float32) for _ in range(_NBUF)],
            pltpu.VMEM_SHARED((_N, _D), jnp.float32),
            [pltpu.SemaphoreType.DMA for _ in range(_NBUF)],
            [pltpu.SemaphoreType.DMA for _ in range(_NBUF)],
            [pltpu.SemaphoreType.DMA for _ in range(_NBUF)],
        ],
    )
    return f(x, src0, dst0, src1, dst1)


def _mlp_body(h0, h1, W10, b10, g0, be0, W11, b11, g1, be1, W2cat, b2sum,
              out):
    def mlp1(h_ref, W1, b1, g, be):
        t = jnp.dot(h_ref[...], W1[...], preferred_element_type=jnp.float32)
        t = t + b1[...]
        m = jnp.mean(t, axis=0, keepdims=True)
        ct = t - m
        v = jnp.mean(ct * ct, axis=0, keepdims=True)
        n = ct * lax.rsqrt(v + 1e-5) * g[...] + be[...]
        return jnp.maximum(n, 0.0)

    r = jnp.concatenate((mlp1(h0, W10, b10, g0, be0),
                         mlp1(h1, W11, b11, g1, be1)), axis=1)
    out[...] = (jnp.dot(r, W2cat[...], preferred_element_type=jnp.float32)
                + b2sum[...])


@jax.jit
def _mlp(h0, h1, *params):
    return pl.pallas_call(
        _mlp_body,
        out_shape=jax.ShapeDtypeStruct((_N, _D), jnp.float32),
    )(h0, h1, *params)


def kernel(x, edge_index_e0, edge_index_e1,
           W1_e0, b1_e0, gamma_e0, beta_e0, W2_e0, b2_e0,
           W1_e1, b1_e1, gamma_e1, beta_e1, W2_e1, b2_e1):
    h0, h1 = _sc_segment(x, edge_index_e0[0], edge_index_e0[1],
                         edge_index_e1[0], edge_index_e1[1])

    p2 = lambda a: a.reshape(1, _D)
    W2cat = jnp.concatenate((W2_e0, W2_e1), axis=0)
    b2sum = p2(b2_e0 + b2_e1)
    return _mlp(h0, h1,
                W1_e0, p2(b1_e0), p2(gamma_e0), p2(beta_e0),
                W1_e1, p2(b1_e1), p2(gamma_e1), p2(beta_e1),
                W2cat, b2sum)
